# baseline (speedup 1.0000x reference)
import functools

import jax
import jax.numpy as jnp
from jax import lax
from jax.experimental import pallas as pl
from jax.experimental.pallas import tpu as pltpu
from jax.experimental.pallas import tpu_sc as plsc

B = 16384
EMB = 64
NCATE = 5
CATN = 40
NC = 2
NS = 16
NW = NC * NS
RPW = B // NW
CH = 128
NCH = RPW // CH

_mesh = plsc.VectorSubcoreMesh(core_axis_name="c", subcore_axis_name="s")


@functools.partial(
    pl.kernel,
    out_type=jax.ShapeDtypeStruct((B, 3 * EMB), jnp.float32),
    mesh=_mesh,
    compiler_params=pltpu.CompilerParams(use_tc_tiling_on_sc=False),
    scratch_types=[
        pltpu.VMEM((NCH, CH), jnp.int32),
        pltpu.VMEM((NCH, CH), jnp.int32),
        pltpu.VMEM((RPW * NCATE + 16,), jnp.int32),
        pltpu.VMEM((RPW, EMB), jnp.float32),
        pltpu.VMEM((RPW, EMB), jnp.float32),
        pltpu.VMEM((RPW, EMB), jnp.float32),
        pltpu.VMEM((CATN * EMB,), jnp.float32),
        pltpu.SemaphoreType.DMA,
        pltpu.SemaphoreType.DMA,
    ],
)
def _din_lookup(uid, bid, cid, utab, btab, ctab, out,
                uidx_v, bidx_v, cidx_v, urows, brows, crows, ctab_v,
                gsem, wsem):
    c = lax.axis_index("c")
    s = lax.axis_index("s")
    wid = s * NC + c
    base = wid * RPW

    pltpu.sync_copy(uid.at[pl.ds(wid * NCH, NCH), :], uidx_v)
    pltpu.sync_copy(bid.at[pl.ds(wid * NCH, NCH), :], bidx_v)
    pltpu.sync_copy(cid.at[wid], cidx_v.at[pl.ds(0, RPW * NCATE)])
    pltpu.sync_copy(ctab, ctab_v)

    gathers = []
    for j in range(NCH):
        gathers.append(pltpu.async_copy(
            utab.at[uidx_v.at[j]], urows.at[pl.ds(j * CH, CH), :], gsem))
    for j in range(NCH):
        gathers.append(pltpu.async_copy(
            btab.at[bidx_v.at[j]], brows.at[pl.ds(j * CH, CH), :], gsem))

    def cate_row(i, carry):
        iv = cidx_v[pl.ds(i * NCATE, 16)]
        offs = [iv[j] * EMB for j in range(NCATE)]
        for q in range(EMB // 16):
            acc = ctab_v[pl.ds(offs[0] + q * 16, 16)]
            for j in range(1, NCATE):
                acc = acc + ctab_v[pl.ds(offs[j] + q * 16, 16)]
            crows[i, pl.ds(q * 16, 16)] = acc * (1.0 / NCATE)
        return carry

    lax.fori_loop(0, RPW // 2, cate_row, 0)
    for g in gathers:
        g.wait()
    wu = pltpu.async_copy(urows, out.at[pl.ds(base, RPW), pl.ds(0, EMB)], wsem)
    wb = pltpu.async_copy(brows, out.at[pl.ds(base, RPW), pl.ds(EMB, EMB)], wsem)
    lax.fori_loop(RPW // 2, RPW, cate_row, 0)
    pltpu.sync_copy(crows, out.at[pl.ds(base, RPW), pl.ds(2 * EMB, EMB)])
    wu.wait()
    wb.wait()


def kernel(user_id, block_id, cate_idx, users_table, block_table, category_table):
    uid = user_id.astype(jnp.int32).reshape(NW * NCH, CH)
    bid = block_id.astype(jnp.int32).reshape(NW * NCH, CH)
    cid = cate_idx.astype(jnp.int32).reshape(NW, RPW * NCATE)
    ctab = category_table.reshape(-1)
    return _din_lookup(uid, bid, cid, users_table, block_table, ctab)


# speedup vs baseline: 1.8185x; 1.8185x over previous
import functools

import jax
import jax.numpy as jnp
from jax import lax
from jax.experimental import pallas as pl
from jax.experimental.pallas import tpu as pltpu
from jax.experimental.pallas import tpu_sc as plsc

B = 16384
EMB = 64
NCATE = 5
CATN = 40
BLKN = 100
NC = 2
NS = 16
NW = NC * NS
RPW = B // NW
CH = 64
NCHK = RPW // CH
NB = CH // 16
NQ = EMB // 16

_mesh = plsc.VectorSubcoreMesh(core_axis_name="c", subcore_axis_name="s")


@functools.partial(
    pl.kernel,
    out_type=(
        jax.ShapeDtypeStruct((B, EMB), jnp.float32),
        jax.ShapeDtypeStruct((B, EMB), jnp.float32),
        jax.ShapeDtypeStruct((B, EMB), jnp.float32),
    ),
    mesh=_mesh,
    scratch_types=[
        pltpu.VMEM((RPW,), jnp.int32),
        pltpu.VMEM((RPW,), jnp.int32),
        pltpu.VMEM((RPW * NCATE + 16,), jnp.int32),
        pltpu.VMEM((2, 16, 8, EMB), jnp.float32),
        pltpu.VMEM((2, CH, EMB), jnp.float32),
        pltpu.VMEM((2, CH, EMB), jnp.float32),
        pltpu.VMEM((2, CH, EMB), jnp.float32),
        pltpu.VMEM((BLKN * EMB,), jnp.float32),
        pltpu.VMEM((CATN * EMB,), jnp.float32),
        pltpu.SemaphoreType.DMA,
        pltpu.SemaphoreType.DMA,
        pltpu.SemaphoreType.DMA,
    ],
)
def _din_lookup(uid, bid, cid, utab3, btab, ctab, uout, bout, cout,
                uidx_v, bidx_v, cidx_v, tiles_v, ubuf, bbuf, cbuf,
                btab_v, ctab_v, semA, semB, wsem):
    c = lax.axis_index("c")
    s = lax.axis_index("s")
    wid = s * NC + c
    base = wid * RPW

    pltpu.sync_copy(uid.at[pl.ds(base, RPW)], uidx_v)
    pltpu.sync_copy(bid.at[pl.ds(base, RPW)], bidx_v)
    pltpu.sync_copy(cid.at[pl.ds(base * NCATE, RPW * NCATE)],
                    cidx_v.at[pl.ds(0, RPW * NCATE)])
    pltpu.sync_copy(btab, btab_v)
    pltpu.sync_copy(ctab, ctab_v)

    def fire(l0, half, sem):
        ivec = uidx_v[pl.ds(l0, 16)]
        for r in range(16):
            tile = lax.shift_right_logical(ivec[r], 3)
            pltpu.async_copy(utab3.at[tile], tiles_v.at[half, r], sem)

    def drain(half, sem):
        for r in range(16):
            pltpu.make_async_copy(utab3.at[0], tiles_v.at[half, r], sem).wait()

    def proc(kb, l0, b0, half):
        ivec = uidx_v[pl.ds(l0, 16)]
        bvec = bidx_v[pl.ds(l0, 16)] * EMB
        for r in range(16):
            r8 = lax.rem(ivec[r], 8)
            civ = cidx_v[pl.ds((l0 + r) * NCATE, 16)]
            offs = [civ[j] * EMB for j in range(NCATE)]
            for q in range(NQ):
                sl = pl.ds(q * 16, 16)
                ubuf[kb, b0 + r, sl] = tiles_v[half, r, r8, sl]
                bbuf[kb, b0 + r, sl] = btab_v[pl.ds(bvec[r] + q * 16, 16)]
                acc = ctab_v[pl.ds(offs[0] + q * 16, 16)]
                for j in range(1, NCATE):
                    acc = acc + ctab_v[pl.ds(offs[j] + q * 16, 16)]
                cbuf[kb, b0 + r, sl] = acc * (1.0 / NCATE)

    def chunk(k, carry):
        kb = lax.rem(k, 2)
        l0 = k * CH
        fire(l0, 0, semA)
        fire(l0 + 16, 1, semB)
        drain(0, semA)
        proc(kb, l0, 0, 0)
        fire(l0 + 32, 0, semA)
        drain(1, semB)
        proc(kb, l0 + 16, 16, 1)
        fire(l0 + 48, 1, semB)
        drain(0, semA)
        proc(kb, l0 + 32, 32, 0)
        drain(1, semB)
        proc(kb, l0 + 48, 48, 1)
        @pl.when(k >= 2)
        def _():
            for buf in (ubuf, bbuf, cbuf):
                pltpu.make_async_copy(
                    buf.at[kb], uout.at[pl.ds(0, CH), :], wsem).wait()
        row0 = base + l0
        pltpu.async_copy(ubuf.at[kb], uout.at[pl.ds(row0, CH), :], wsem)
        pltpu.async_copy(bbuf.at[kb], bout.at[pl.ds(row0, CH), :], wsem)
        pltpu.async_copy(cbuf.at[kb], cout.at[pl.ds(row0, CH), :], wsem)
        return carry

    lax.fori_loop(0, NCHK, chunk, 0)
    for _ in range(2):
        for buf in (ubuf, bbuf, cbuf):
            pltpu.make_async_copy(
                buf.at[0], uout.at[pl.ds(0, CH), :], wsem).wait()


def kernel(user_id, block_id, cate_idx, users_table, block_table, category_table):
    uid = user_id.astype(jnp.int32)
    bid = block_id.astype(jnp.int32)
    cid = cate_idx.astype(jnp.int32).reshape(-1)
    utab3 = users_table.reshape(125000, 8, EMB)
    btab = block_table.reshape(-1)
    ctab = category_table.reshape(-1)
    u, b, cc = _din_lookup(uid, bid, cid, utab3, btab, ctab)
    return jnp.concatenate([u, b, cc], axis=1)


# speedup vs baseline: 1.9414x; 1.0676x over previous
import functools

import jax
import jax.numpy as jnp
from jax import lax
from jax.experimental import pallas as pl
from jax.experimental.pallas import tpu as pltpu
from jax.experimental.pallas import tpu_sc as plsc

B = 16384
EMB = 64
NCATE = 5
CATN = 40
BLKN = 100
NC = 2
NS = 16
NW = NC * NS
RPW = B // NW
CH = 128
NCHK = RPW // CH
NQ = EMB // 16

_mesh = plsc.VectorSubcoreMesh(core_axis_name="c", subcore_axis_name="s")


@functools.partial(
    pl.kernel,
    out_type=(
        jax.ShapeDtypeStruct((B, 2 * EMB), jnp.float32),
        jax.ShapeDtypeStruct((B, EMB), jnp.float32),
    ),
    mesh=_mesh,
    scratch_types=[
        pltpu.VMEM((RPW + 32,), jnp.int32),
        pltpu.VMEM((RPW,), jnp.int32),
        pltpu.VMEM((RPW * NCATE + 16,), jnp.int32),
        pltpu.VMEM((2, 16, 8, EMB), jnp.float32),
        pltpu.VMEM((2, CH, 2 * EMB), jnp.float32),
        pltpu.VMEM((2, CH, EMB), jnp.float32),
        pltpu.VMEM((BLKN * EMB,), jnp.float32),
        pltpu.VMEM((CATN * EMB,), jnp.float32),
        pltpu.SemaphoreType.DMA,
        pltpu.SemaphoreType.DMA,
        pltpu.SemaphoreType.DMA,
    ],
)
def _din_lookup(uid, bid, cid, utab3, btab, ctab, ubout, cout,
                uidx_v, bidx_v, cidx_v, tiles_v, ubuf, cbuf,
                btab_v, ctab_v, semA, semB, wsem):
    c = lax.axis_index("c")
    s = lax.axis_index("s")
    wid = s * NC + c
    base = wid * RPW

    pltpu.sync_copy(uid.at[pl.ds(base, RPW)], uidx_v.at[pl.ds(0, RPW)])
    pltpu.sync_copy(uid.at[pl.ds(base, 32)], uidx_v.at[pl.ds(RPW, 32)])
    pltpu.sync_copy(bid.at[pl.ds(base, RPW)], bidx_v)
    pltpu.sync_copy(cid.at[pl.ds(base * NCATE, RPW * NCATE)],
                    cidx_v.at[pl.ds(0, RPW * NCATE)])
    pltpu.sync_copy(btab, btab_v)
    pltpu.sync_copy(ctab, ctab_v)

    def fire(g, half, sem):
        ivec = uidx_v[pl.ds(g * 16, 16)]
        for r in range(16):
            tile = lax.shift_right_logical(ivec[r], 3)
            pltpu.async_copy(utab3.at[tile], tiles_v.at[half, r], sem)

    def drain(half, sem):
        for r in range(16):
            pltpu.make_async_copy(utab3.at[0], tiles_v.at[half, r], sem).wait()

    def proc(g, kb, b0, half):
        ivec = uidx_v[pl.ds(g * 16, 16)]
        bvec = bidx_v[pl.ds(g * 16, 16)] * EMB
        for r in range(16):
            r8 = lax.rem(ivec[r], 8)
            civ = cidx_v[pl.ds((g * 16 + r) * NCATE, 16)]
            offs = [civ[j] * EMB for j in range(NCATE)]
            for q in range(NQ):
                sl = pl.ds(q * 16, 16)
                ubuf[kb, b0 + r, sl] = tiles_v[half, r, r8, sl]
                ubuf[kb, b0 + r, pl.ds(EMB + q * 16, 16)] = (
                    btab_v[pl.ds(bvec[r] + q * 16, 16)])
                acc = ctab_v[pl.ds(offs[0] + q * 16, 16)]
                for j in range(1, NCATE):
                    acc = acc + ctab_v[pl.ds(offs[j] + q * 16, 16)]
                cbuf[kb, b0 + r, sl] = acc * (1.0 / NCATE)

    fire(0, 0, semA)
    fire(1, 1, semB)

    def chunk(k, carry):
        kb = lax.rem(k, 2)

        @pl.when(k >= 1)
        def _():
            pltpu.make_async_copy(
                ubuf.at[0], ubout.at[pl.ds(0, CH), :], wsem).wait()
            pltpu.make_async_copy(
                cbuf.at[0], cout.at[pl.ds(0, CH), :], wsem).wait()

        def pair(q, carry2):
            g = k * (CH // 16) + 2 * q
            drain(0, semA)
            proc(g, kb, 2 * q * 16, 0)
            fire(g + 2, 0, semA)
            drain(1, semB)
            proc(g + 1, kb, 2 * q * 16 + 16, 1)
            fire(g + 3, 1, semB)
            return carry2

        lax.fori_loop(0, CH // 32, pair, 0)
        row0 = base + k * CH
        pltpu.async_copy(ubuf.at[kb], ubout.at[pl.ds(row0, CH), :], wsem)
        pltpu.async_copy(cbuf.at[kb], cout.at[pl.ds(row0, CH), :], wsem)
        return carry

    lax.fori_loop(0, NCHK, chunk, 0)

    drain(0, semA)
    drain(1, semB)
    pltpu.make_async_copy(ubuf.at[0], ubout.at[pl.ds(0, CH), :], wsem).wait()
    pltpu.make_async_copy(cbuf.at[0], cout.at[pl.ds(0, CH), :], wsem).wait()


def kernel(user_id, block_id, cate_idx, users_table, block_table, category_table):
    uid = user_id.astype(jnp.int32)
    bid = block_id.astype(jnp.int32)
    cid = cate_idx.astype(jnp.int32).reshape(-1)
    utab3 = users_table.reshape(125000, 8, EMB)
    btab = block_table.reshape(-1)
    ctab = category_table.reshape(-1)
    ub, cc = _din_lookup(uid, bid, cid, utab3, btab, ctab)
    return jnp.concatenate([ub, cc], axis=1)
